# Initial kernel scaffold; baseline (speedup 1.0000x reference)
import jax
import jax.numpy as jnp
from jax.experimental import pallas as pl


def kernel(x, edge_index, W_l1, b_l1, W_c1, b_c1, W_c2, b_c2):
    raise NotImplementedError("write your pallas kernel here")


# speedup vs baseline: 9.4671x; 9.4671x over previous
import functools

import jax
import jax.numpy as jnp
from jax import lax
from jax.experimental import pallas as pl
from jax.experimental.pallas import tpu as pltpu
from jax.experimental.pallas import tpu_sc as plsc

N = 10000
E = 320000
D = 128
DO = 64
MU = 0.1
ALPHA = 1.0 / (1.0 + MU)
BETA = MU / (1.0 + MU)

NC = 2
NS = 16
NW = NC * NS
E_TILE = E // NW
EB = 80
NB = E_TILE // EB
N_P = 10240
SLN = N_P // NS
N_AGG = N_P
ROWS_T = N_AGG // NS
RB = 128

_mesh = plsc.VectorSubcoreMesh(core_axis_name="c", subcore_axis_name="s")

_GDN = lax.GatherDimensionNumbers(
    offset_dims=(), collapsed_slice_dims=(0,), start_index_map=(0,))


def _bcast_lane(vec, l):
    idx = jnp.full((16, 1), l, jnp.int32)
    return lax.gather(vec, idx, _GDN, slice_sizes=(1,),
                      mode=lax.GatherScatterMode.PROMISE_IN_BOUNDS)


def _zero_vec_ref(ref, n):
    for j in range(n // 16):
        ref[pl.ds(j * 16, 16)] = jnp.zeros((16,), jnp.float32)


@functools.partial(
    pl.kernel,
    out_type=jax.ShapeDtypeStruct((NC, N_P), jnp.float32),
    mesh=_mesh,
    compiler_params=pltpu.CompilerParams(needs_layout_passes=False),
    scratch_types=[
        pltpu.VMEM((EB,), jnp.int32),
        pltpu.VMEM((EB,), jnp.float32),
        pltpu.VMEM((SLN,), jnp.float32),
        pltpu.VMEM_SHARED((N_P,), jnp.float32),
    ],
)
def _sc_degree(dst_hbm, out_hbm, idx_v, ones_v, zbuf_v, acc_sh):
    c = lax.axis_index("c")
    s = lax.axis_index("s")
    _zero_vec_ref(zbuf_v, SLN)
    pltpu.sync_copy(zbuf_v, acc_sh.at[pl.ds(s * SLN, SLN)])
    for j in range(EB // 16):
        ones_v[pl.ds(j * 16, 16)] = jnp.ones((16,), jnp.float32)
    plsc.subcore_barrier()
    base = (c * NS + s) * E_TILE

    def body(i, carry):
        pltpu.sync_copy(dst_hbm.at[pl.ds(base + i * EB, EB)], idx_v)
        pltpu.sync_copy(ones_v, acc_sh.at[idx_v], add=True)
        return carry

    lax.fori_loop(0, NB, body, 0)
    plsc.subcore_barrier()
    pltpu.sync_copy(acc_sh.at[pl.ds(s * SLN, SLN)],
                    out_hbm.at[c, pl.ds(s * SLN, SLN)])


@functools.partial(
    pl.kernel,
    out_type=jax.ShapeDtypeStruct((NC, N_P), jnp.float32),
    mesh=_mesh,
    compiler_params=pltpu.CompilerParams(needs_layout_passes=False),
    scratch_types=[
        pltpu.VMEM((EB,), jnp.int32),
        pltpu.VMEM((EB,), jnp.int32),
        pltpu.VMEM((EB,), jnp.float32),
        pltpu.VMEM((N_P,), jnp.float32),
        pltpu.VMEM((SLN,), jnp.float32),
        pltpu.VMEM_SHARED((N_P,), jnp.float32),
    ],
)
def _sc_dsum(src_hbm, dst_hbm, dinv_hbm, out_hbm,
             idx_s, idx_d, ew_v, dinv_v, zbuf_v, acc_sh):
    c = lax.axis_index("c")
    s = lax.axis_index("s")
    _zero_vec_ref(zbuf_v, SLN)
    pltpu.sync_copy(zbuf_v, acc_sh.at[pl.ds(s * SLN, SLN)])
    pltpu.sync_copy(dinv_hbm, dinv_v)
    plsc.subcore_barrier()
    base = (c * NS + s) * E_TILE

    def body(i, carry):
        pltpu.sync_copy(src_hbm.at[pl.ds(base + i * EB, EB)], idx_s)
        pltpu.sync_copy(dst_hbm.at[pl.ds(base + i * EB, EB)], idx_d)
        for g in range(EB // 16):
            sl = pl.ds(g * 16, 16)
            vs = plsc.load_gather(dinv_v, [idx_s[sl]])
            vd = plsc.load_gather(dinv_v, [idx_d[sl]])
            ew_v[sl] = vs * vd
        pltpu.sync_copy(ew_v, acc_sh.at[idx_s], add=True)
        return carry

    lax.fori_loop(0, NB, body, 0)
    plsc.subcore_barrier()
    pltpu.sync_copy(acc_sh.at[pl.ds(s * SLN, SLN)],
                    out_hbm.at[c, pl.ds(s * SLN, SLN)])


@functools.partial(
    pl.kernel,
    out_type=jax.ShapeDtypeStruct((E,), jnp.float32),
    mesh=_mesh,
    compiler_params=pltpu.CompilerParams(needs_layout_passes=False),
    scratch_types=[
        pltpu.VMEM((EB,), jnp.int32),
        pltpu.VMEM((EB,), jnp.int32),
        pltpu.VMEM((EB,), jnp.float32),
        pltpu.VMEM((N_P,), jnp.float32),
    ],
)
def _sc_coef(src_hbm, dst_hbm, q_hbm, out_hbm, idx_s, idx_d, cf_v, q_v):
    c = lax.axis_index("c")
    s = lax.axis_index("s")
    pltpu.sync_copy(q_hbm, q_v)
    base = (c * NS + s) * E_TILE

    def body(i, carry):
        pltpu.sync_copy(src_hbm.at[pl.ds(base + i * EB, EB)], idx_s)
        pltpu.sync_copy(dst_hbm.at[pl.ds(base + i * EB, EB)], idx_d)
        for g in range(EB // 16):
            sl = pl.ds(g * 16, 16)
            vs = plsc.load_gather(q_v, [idx_s[sl]])
            vd = plsc.load_gather(q_v, [idx_d[sl]])
            cf_v[sl] = vs * vd
        pltpu.sync_copy(cf_v, out_hbm.at[pl.ds(base + i * EB, EB)])
        return carry

    lax.fori_loop(0, NB, body, 0)


@functools.partial(
    pl.kernel,
    out_type=jax.ShapeDtypeStruct((NC, N_AGG, D), jnp.float32),
    mesh=_mesh,
    compiler_params=pltpu.CompilerParams(needs_layout_passes=False),
    scratch_types=[
        pltpu.VMEM((EB,), jnp.int32),
        pltpu.VMEM((EB,), jnp.int32),
        pltpu.VMEM((EB,), jnp.float32),
        pltpu.VMEM((EB, D), jnp.float32),
        pltpu.VMEM((RB, D), jnp.float32),
        pltpu.VMEM_SHARED((N_AGG, D), jnp.float32),
        pltpu.SemaphoreType.DMA,
    ],
)
def _sc_spmv(src_hbm, dst_hbm, coef_hbm, f_hbm, out_hbm,
             idx_s, idx_d, cf_v, rows_v, zbuf_v, acc_sh, sem):
    c = lax.axis_index("c")
    s = lax.axis_index("s")
    nb = s * ROWS_T

    def zrow(r, carry):
        for j in range(D // 16):
            zbuf_v[r, pl.ds(j * 16, 16)] = jnp.zeros((16,), jnp.float32)
        return carry

    lax.fori_loop(0, RB, zrow, 0)
    for k in range(ROWS_T // RB):
        pltpu.sync_copy(zbuf_v, acc_sh.at[pl.ds(nb + k * RB, RB)])
    plsc.subcore_barrier()
    base = (c * NS + s) * E_TILE

    def body(i, carry):
        pltpu.sync_copy(src_hbm.at[pl.ds(base + i * EB, EB)], idx_s)
        pltpu.sync_copy(dst_hbm.at[pl.ds(base + i * EB, EB)], idx_d)
        pltpu.sync_copy(coef_hbm.at[pl.ds(base + i * EB, EB)], cf_v)
        pltpu.async_copy(f_hbm.at[idx_d], rows_v, sem).wait()
        for g in range(EB // 16):
            cv = cf_v[pl.ds(g * 16, 16)]
            for l in range(16):
                r = g * 16 + l
                cb = _bcast_lane(cv, l)
                for j in range(D // 16):
                    sl = pl.ds(j * 16, 16)
                    rows_v[r, sl] = rows_v[r, sl] * cb
        pltpu.sync_copy(rows_v, acc_sh.at[idx_s], add=True)
        return carry

    lax.fori_loop(0, NB, body, 0)
    plsc.subcore_barrier()
    for k in range(ROWS_T // RB):
        pltpu.sync_copy(acc_sh.at[pl.ds(nb + k * RB, RB)],
                        out_hbm.at[c, pl.ds(nb + k * RB, RB)])


def _tc_dinv_body(deg_ref, out_ref):
    out_ref[...] = lax.rsqrt(deg_ref[0] + deg_ref[1] + 1.0)


def _tc_dinv(deg2):
    return pl.pallas_call(
        _tc_dinv_body,
        out_shape=jax.ShapeDtypeStruct((N_P // D, D), jnp.float32),
    )(deg2.reshape(2, N_P // D, D))


def _tc_q_body(d2_ref, dinv_ref, q_ref, q2_ref):
    dinv = dinv_ref[...]
    dd = jnp.maximum(d2_ref[0] + d2_ref[1] + dinv * dinv, 1e-12)
    q = dinv * lax.rsqrt(dd)
    q_ref[...] = q
    q2_ref[...] = q * q


def _tc_q(d2, dinv):
    return pl.pallas_call(
        _tc_q_body,
        out_shape=(
            jax.ShapeDtypeStruct((N_P // D, D), jnp.float32),
            jax.ShapeDtypeStruct((N_P // D, D), jnp.float32),
        ),
    )(d2.reshape(2, N_P // D, D), dinv)


_RB_TC = 2000


def _tc_linrelu_body(x_ref, w_ref, b_ref, out_ref):
    y = jnp.dot(x_ref[...], w_ref[...], preferred_element_type=jnp.float32)
    out_ref[...] = jnp.maximum(y + b_ref[...], 0.0)


def _tc_linrelu(x, w, b):
    return pl.pallas_call(
        _tc_linrelu_body,
        out_shape=jax.ShapeDtypeStruct((N, D), jnp.float32),
        grid=(N // _RB_TC,),
        in_specs=[
            pl.BlockSpec((_RB_TC, D), lambda i: (i, 0)),
            pl.BlockSpec((D, D), lambda i: (0, 0)),
            pl.BlockSpec((1, D), lambda i: (0, 0)),
        ],
        out_specs=pl.BlockSpec((_RB_TC, D), lambda i: (i, 0)),
    )(x, w, b.reshape(1, D))


def _combine(g0, g1, f, f0, q2):
    return ALPHA * (g0 + g1 + q2 * f) + BETA * f0


def _tc_combine_body(g0_ref, g1_ref, f_ref, f0_ref, q2_ref, out_ref):
    out_ref[...] = _combine(g0_ref[...], g1_ref[...], f_ref[...],
                            f0_ref[...], q2_ref[...])


def _tc_combine(g0, g1, f, f0, q2):
    return pl.pallas_call(
        _tc_combine_body,
        out_shape=jax.ShapeDtypeStruct((N, D), jnp.float32),
        grid=(N // _RB_TC,),
        in_specs=[
            pl.BlockSpec((_RB_TC, D), lambda i: (i, 0)),
            pl.BlockSpec((_RB_TC, D), lambda i: (i, 0)),
            pl.BlockSpec((_RB_TC, D), lambda i: (i, 0)),
            pl.BlockSpec((_RB_TC, D), lambda i: (i, 0)),
            pl.BlockSpec((_RB_TC, 1), lambda i: (i, 0)),
        ],
        out_specs=pl.BlockSpec((_RB_TC, D), lambda i: (i, 0)),
    )(g0, g1, f, f0, q2)


def _tc_comb_mm_relu_body(g0_ref, g1_ref, f_ref, f0_ref, q2_ref, w_ref, b_ref,
                          out_ref):
    z = _combine(g0_ref[...], g1_ref[...], f_ref[...], f0_ref[...], q2_ref[...])
    y = jnp.dot(z, w_ref[...], preferred_element_type=jnp.float32)
    out_ref[...] = jnp.maximum(y + b_ref[...], 0.0)


def _tc_comb_mm_relu(g0, g1, f, f0, q2, w, b):
    return pl.pallas_call(
        _tc_comb_mm_relu_body,
        out_shape=jax.ShapeDtypeStruct((N, D), jnp.float32),
        grid=(N // _RB_TC,),
        in_specs=[
            pl.BlockSpec((_RB_TC, D), lambda i: (i, 0)),
            pl.BlockSpec((_RB_TC, D), lambda i: (i, 0)),
            pl.BlockSpec((_RB_TC, D), lambda i: (i, 0)),
            pl.BlockSpec((_RB_TC, D), lambda i: (i, 0)),
            pl.BlockSpec((_RB_TC, 1), lambda i: (i, 0)),
            pl.BlockSpec((D, D), lambda i: (0, 0)),
            pl.BlockSpec((1, D), lambda i: (0, 0)),
        ],
        out_specs=pl.BlockSpec((_RB_TC, D), lambda i: (i, 0)),
    )(g0, g1, f, f0, q2, w, b.reshape(1, D))


def _tc_comb_mm_lsm_body(g0_ref, g1_ref, f_ref, f0_ref, q2_ref, w_ref, b_ref,
                         out_ref):
    z = _combine(g0_ref[...], g1_ref[...], f_ref[...], f0_ref[...], q2_ref[...])
    y = jnp.dot(z, w_ref[...], preferred_element_type=jnp.float32) + b_ref[...]
    m = jnp.max(y, axis=1, keepdims=True)
    lse = m + jnp.log(jnp.sum(jnp.exp(y - m), axis=1, keepdims=True))
    out_ref[...] = y - lse


def _tc_comb_mm_lsm(g0, g1, f, f0, q2, w, b):
    return pl.pallas_call(
        _tc_comb_mm_lsm_body,
        out_shape=jax.ShapeDtypeStruct((N, DO), jnp.float32),
        grid=(N // _RB_TC,),
        in_specs=[
            pl.BlockSpec((_RB_TC, D), lambda i: (i, 0)),
            pl.BlockSpec((_RB_TC, D), lambda i: (i, 0)),
            pl.BlockSpec((_RB_TC, D), lambda i: (i, 0)),
            pl.BlockSpec((_RB_TC, D), lambda i: (i, 0)),
            pl.BlockSpec((_RB_TC, 1), lambda i: (i, 0)),
            pl.BlockSpec((D, DO), lambda i: (0, 0)),
            pl.BlockSpec((1, DO), lambda i: (0, 0)),
        ],
        out_specs=pl.BlockSpec((_RB_TC, DO), lambda i: (i, 0)),
    )(g0, g1, f, f0, q2, w, b.reshape(1, DO))


def kernel(x, edge_index, W_l1, b_l1, W_c1, b_c1, W_c2, b_c2):
    src = edge_index[0].astype(jnp.int32)
    dst = edge_index[1].astype(jnp.int32)

    deg2 = _sc_degree(dst)
    dinv = _tc_dinv(deg2)
    d2 = _sc_dsum(src, dst, dinv.reshape(N_P))
    q, q2f = _tc_q(d2, dinv)
    coef = _sc_coef(src, dst, q.reshape(N_P))
    q2 = q2f.reshape(N_P)[:N].reshape(N, 1)

    f0 = _tc_linrelu(x, W_l1, b_l1)

    g = _sc_spmv(src, dst, coef, f0)
    f1 = _tc_combine(g[0], g[1], f0, f0, q2)
    g = _sc_spmv(src, dst, coef, f1)
    h = _tc_comb_mm_relu(g[0], g[1], f1, f0, q2, W_c1, b_c1)

    g = _sc_spmv(src, dst, coef, h)
    f1 = _tc_combine(g[0], g[1], h, h, q2)
    g = _sc_spmv(src, dst, coef, f1)
    out = _tc_comb_mm_lsm(g[0], g[1], f1, h, q2, W_c2, b_c2)
    return out
